# baseline (speedup 1.0000x reference)
import jax
import jax.numpy as jnp
from jax.experimental import pallas as pl
from jax.experimental.pallas import tpu as pltpu

_P = 16
_TS = 12
_L = 64
_K = 4096
_EOT = 2
_EOI = 32001
_OFFSET = 32002


def _patchify_kernel(img_ref, x_ref, sem):
    C = img_ref.shape[1]
    G = img_ref.shape[2]
    nb = pl.num_programs(0)
    b = pl.program_id(0)

    def copies(i):
        return [
            pltpu.make_async_copy(
                img_ref.at[i, c, :, :, gw, :],
                x_ref.at[i, pl.ds(gw * _P, _P), c, :, :],
                sem,
            )
            for c in range(C) for gw in range(G)
        ]

    @pl.when(b == 0)
    def _():
        for cp in copies(0):
            cp.start()

    @pl.when(b + 1 < nb)
    def _():
        for cp in copies(b + 1):
            cp.start()

    for cp in copies(b):
        cp.wait()


def _vq_kernel(x_ref, wm_ref, wp_ref, bp_ref, wproj_ref, cb_ref, flag_ref,
               out_ref, wc_s, sbb_s, cbn_s, cn2_s):
    @pl.when(pl.program_id(0) == 0)
    def _():
        wc_s[...] = jax.lax.dot_general(
            wp_ref[...], wproj_ref[...], (((1,), (0,)), ((), ())))
        bb = jax.lax.dot_general(
            bp_ref[...], wproj_ref[...], (((1,), (0,)), ((), ())))
        ones_p = jnp.ones((1, wm_ref.shape[0]), jnp.float32)
        s_col = jax.lax.dot_general(
            wm_ref[...], ones_p, (((0,), (1,)), ((), ())))
        sbb_s[...] = s_col * bb
        cb = cb_ref[...]
        nrm = jnp.sqrt(jnp.sum(cb * cb, axis=1, keepdims=True))
        cbn = cb / (nrm + 1e-6)
        cbn_s[...] = cbn
        ones_t = jnp.ones((1, cb.shape[1]), jnp.float32)
        cn2_s[...] = jax.lax.dot_general(
            ones_t, cbn * cbn, (((1,), (1,)), ((), ())))

    x = x_ref[0]
    mix = jax.lax.dot_general(wm_ref[...], x, (((0,), (0,)), ((), ())))
    z = jax.lax.dot_general(
        mix, wc_s[...], (((1,), (0,)), ((), ()))) + sbb_s[...]
    zn = z / (jnp.sqrt(jnp.sum(z * z, axis=1, keepdims=True)) + 1e-6)
    dots = jax.lax.dot_general(zn, cbn_s[...], (((1,), (1,)), ((), ())))
    scores = cn2_s[...] - 2.0 * dots
    idx = jnp.argmin(scores, axis=1).astype(jnp.int32)
    flag = flag_ref[0]
    row = jnp.concatenate(
        [(idx + _OFFSET)[None, :],
         jnp.full((1, 1), _EOI, jnp.int32),
         flag * (jax.lax.broadcasted_iota(jnp.int32, (1, 2), 1) + _EOT - 1)],
        axis=1)
    out_ref[0] = row


def kernel(image, append_empty_text, W_patch, b_patch, W_mix, W_proj, codebook):
    B, C, H, _ = image.shape
    G = H // _P
    NP = G * G
    D = W_patch.shape[1]
    PP = _P * _P
    img6 = image.reshape(B, C, G, _P, G, _P)
    wmix_p = W_mix.reshape(G, G, _L).transpose(1, 0, 2).reshape(NP, _L)
    flag = jnp.asarray(append_empty_text).astype(jnp.int32).reshape(1)

    x5 = pl.pallas_call(
        _patchify_kernel,
        grid=(B,),
        in_specs=[pl.BlockSpec(memory_space=pl.ANY)],
        out_specs=pl.BlockSpec(memory_space=pl.ANY),
        out_shape=jax.ShapeDtypeStruct((B, NP, C, _P, _P), jnp.float32),
        scratch_shapes=[pltpu.SemaphoreType.DMA],
        compiler_params=pltpu.CompilerParams(
            dimension_semantics=("arbitrary",)),
    )(img6)
    x = x5.reshape(B, NP, C * PP)

    out = pl.pallas_call(
        _vq_kernel,
        grid=(B,),
        in_specs=[
            pl.BlockSpec((1, NP, C * PP), lambda b: (b, 0, 0)),
            pl.BlockSpec((NP, _L), lambda b: (0, 0)),
            pl.BlockSpec((C * PP, D), lambda b: (0, 0)),
            pl.BlockSpec((1, D), lambda b: (0, 0)),
            pl.BlockSpec((D, _TS), lambda b: (0, 0)),
            pl.BlockSpec((_K, _TS), lambda b: (0, 0)),
            pl.BlockSpec(memory_space=pltpu.SMEM),
        ],
        out_specs=pl.BlockSpec((1, 1, _L + 3), lambda b: (b, 0, 0)),
        out_shape=jax.ShapeDtypeStruct((B, 1, _L + 3), jnp.int32),
        scratch_shapes=[
            pltpu.VMEM((C * PP, _TS), jnp.float32),
            pltpu.VMEM((_L, _TS), jnp.float32),
            pltpu.VMEM((_K, _TS), jnp.float32),
            pltpu.VMEM((1, _K), jnp.float32),
        ],
        compiler_params=pltpu.CompilerParams(
            dimension_semantics=("arbitrary",)),
    )(x, wmix_p, W_patch, b_patch.reshape(1, D), W_proj, codebook, flag)
    return out.reshape(B, _L + 3)


# speedup vs baseline: 59.3916x; 59.3916x over previous
import jax
import jax.numpy as jnp
from jax.experimental import pallas as pl
from jax.experimental.pallas import tpu as pltpu

_P = 16
_TS = 12
_L = 64
_K = 4096
_EOT = 2
_EOI = 32001
_OFFSET = 32002
_NB = 4


def _vq_kernel(x_ref, wm_ref, wp_ref, bp_ref, wproj_ref, cb_ref, flag_ref,
               out_ref, wc_s, sbb_s, cbn_s, cn2_s):
    @pl.when(pl.program_id(0) == 0)
    def _():
        wc_s[...] = jax.lax.dot_general(
            wp_ref[...], wproj_ref[...], (((1,), (0,)), ((), ())))
        bb = jax.lax.dot_general(
            bp_ref[...], wproj_ref[...], (((1,), (0,)), ((), ())))
        ones_p = jnp.ones((1, wm_ref.shape[0]), jnp.float32)
        s_col = jax.lax.dot_general(
            wm_ref[...], ones_p, (((0,), (1,)), ((), ())))
        sbb_s[...] = s_col * bb
        cb = cb_ref[...]
        nrm = jnp.sqrt(jnp.sum(cb * cb, axis=1, keepdims=True))
        cbn = cb / (nrm + 1e-6)
        cbn_s[...] = cbn
        ones_t = jnp.ones((1, cb.shape[1]), jnp.float32)
        cn2_s[...] = jax.lax.dot_general(
            ones_t, cbn * cbn, (((1,), (1,)), ((), ())))

    zs = []
    for j in range(_NB):
        xj = x_ref[j]
        mj = jax.lax.dot_general(
            wm_ref[...], xj, (((0,), (0,)), ((), ())))
        zs.append(jax.lax.dot_general(
            mj, wc_s[...], (((1,), (0,)), ((), ()))) + sbb_s[...])
    z = jnp.concatenate(zs, axis=0)
    zn = z / (jnp.sqrt(jnp.sum(z * z, axis=1, keepdims=True)) + 1e-6)
    dots = jax.lax.dot_general(zn, cbn_s[...], (((1,), (1,)), ((), ())))
    scores = cn2_s[...] - 2.0 * dots
    idx = jnp.argmin(scores, axis=1).astype(jnp.int32)
    flag = flag_ref[0]
    rows = jnp.concatenate(
        [idx.reshape(_NB, _L) + _OFFSET,
         jnp.full((_NB, 1), _EOI, jnp.int32),
         flag * jnp.broadcast_to(
             jax.lax.broadcasted_iota(jnp.int32, (1, 2), 1) + _EOT - 1,
             (_NB, 2))],
        axis=1)
    out_ref[...] = rows.reshape(_NB, 1, _L + 3)


def kernel(image, append_empty_text, W_patch, b_patch, W_mix, W_proj, codebook):
    B, C, H, _ = image.shape
    G = H // _P
    NP = G * G
    D = W_patch.shape[1]
    x = image.reshape(B, C, G, _P, G, _P)
    x = x.transpose(0, 2, 4, 1, 3, 5).reshape(B, NP, C * _P * _P)
    flag = jnp.asarray(append_empty_text).astype(jnp.int32).reshape(1)

    out = pl.pallas_call(
        _vq_kernel,
        grid=(B // _NB,),
        in_specs=[
            pl.BlockSpec((_NB, NP, C * _P * _P), lambda b: (b, 0, 0)),
            pl.BlockSpec((NP, _L), lambda b: (0, 0)),
            pl.BlockSpec((C * _P * _P, D), lambda b: (0, 0)),
            pl.BlockSpec((1, D), lambda b: (0, 0)),
            pl.BlockSpec((D, _TS), lambda b: (0, 0)),
            pl.BlockSpec((_K, _TS), lambda b: (0, 0)),
            pl.BlockSpec(memory_space=pltpu.SMEM),
        ],
        out_specs=pl.BlockSpec((_NB, 1, _L + 3), lambda b: (b, 0, 0)),
        out_shape=jax.ShapeDtypeStruct((B, 1, _L + 3), jnp.int32),
        scratch_shapes=[
            pltpu.VMEM((C * _P * _P, _TS), jnp.float32),
            pltpu.VMEM((_L, _TS), jnp.float32),
            pltpu.VMEM((_K, _TS), jnp.float32),
            pltpu.VMEM((1, _K), jnp.float32),
        ],
        compiler_params=pltpu.CompilerParams(
            dimension_semantics=("arbitrary",)),
    )(x, W_mix, W_patch, b_patch.reshape(1, D), W_proj, codebook, flag)
    return out.reshape(B, _L + 3)
